# baseline (speedup 1.0000x reference)
import functools

import jax
import jax.numpy as jnp
from jax import lax
from jax.experimental import pallas as pl
from jax.experimental.pallas import tpu as pltpu
from jax.experimental.pallas import tpu_sc as plsc

_NC = 2
_NS = 16
_NW = _NC * _NS
_C = 128
_L = 16


@functools.partial(jax.jit, static_argnums=(2,))
def _gather_t(cap_t, table, nseq):
    d = table.shape[1]
    b = cap_t.shape[1]
    mesh = plsc.VectorSubcoreMesh(core_axis_name="c", subcore_axis_name="s")

    @functools.partial(
        pl.kernel,
        out_type=jax.ShapeDtypeStruct((nseq, d, b), jnp.float32),
        mesh=mesh,
        scratch_types=[
            pltpu.VMEM((nseq, _C), jnp.int32),
            pltpu.VMEM((2, _C, d), jnp.float32),
            pltpu.VMEM((2, d, _C), jnp.float32),
            pltpu.SemaphoreType.DMA,
            pltpu.SemaphoreType.DMA,
        ],
        compiler_params=pltpu.CompilerParams(needs_layout_passes=False),
    )
    def k(cap_hbm, tab_hbm, out_hbm, idx_v, rows_v, tile_v, g0, g1):
        wid = lax.axis_index("s") * _NC + lax.axis_index("c")
        b0 = wid * _C
        gsems = (g0, g1)
        pltpu.sync_copy(cap_hbm.at[:, pl.ds(b0, _C)], idx_v)
        lane = lax.iota(jnp.int32, _L)

        def fire(s, buf):
            @pl.loop(0, _C, step=_L)
            def _(i0):
                vec = idx_v[s, pl.ds(i0, _L)]
                for i in range(_L):
                    pltpu.async_copy(
                        tab_hbm.at[vec[i]],
                        rows_v.at[buf, i0 + i],
                        gsems[buf],
                    )

        def wait_gather(buf):
            pltpu.make_async_copy(
                tab_hbm.at[pl.ds(0, _C)], rows_v.at[buf], gsems[buf]
            ).wait()

        def transpose(buf):
            @pl.loop(0, d, step=1)
            def _(dd):
                col = jnp.full((_L,), dd, jnp.int32)
                for i0 in range(0, _C, _L):
                    v = plsc.load_gather(rows_v.at[buf], [lane + i0, col])
                    tile_v[buf, dd, pl.ds(i0, _L)] = v

        def store(s, buf):
            pltpu.sync_copy(tile_v.at[buf], out_hbm.at[s, :, pl.ds(b0, _C)])

        fire(0, 0)
        fire(1, 1)

        @pl.loop(0, nseq - 2, step=2)
        def _(jj):
            for buf in range(2):
                s = jj + buf
                wait_gather(buf)
                transpose(buf)
                fire(s + 2, buf)
                store(s, buf)

        for buf in range(2):
            s = nseq - 2 + buf
            wait_gather(buf)
            transpose(buf)
            store(s, buf)

    return k(cap_t, table)


def kernel(captions, weights):
    bsz, seq = captions.shape
    cap_t = captions.T.astype(jnp.int32)
    out_t = _gather_t(cap_t, weights, seq)
    return out_t.transpose(2, 0, 1)
